# baseline (speedup 1.0000x reference)
import jax
import jax.numpy as jnp
from jax.experimental import pallas as pl
from jax.experimental.pallas import tpu as pltpu

_D = 768
_K = 1024
_TILE_T = 512


def _vq_block(x_ref, w_ref, b_ref, cb_ref, out_ref, s_ref, xn_ref):
    i = pl.program_id(0)
    cb = cb_ref[...]
    slot = jax.lax.rem(i, 2)
    prev = jax.lax.rem(i + 1, 2)

    s = s_ref[prev]
    xn = xn_ref[prev]
    cbn = jnp.sum(cb * cb, axis=1, keepdims=True)
    dist = (xn - 2.0 * s) + cbn
    out_ref[...] = jnp.argmin(dist, axis=0)[None, :].astype(jnp.int32)

    x = x_ref[...]
    w = w_ref[...]
    y = jnp.dot(w, x, preferred_element_type=jnp.float32) + b_ref[...]
    s_ref[slot] = jnp.dot(cb, y, preferred_element_type=jnp.float32)
    xn_ref[slot] = jnp.sum(y * y, axis=0, keepdims=True)


def kernel(ssl_content, proj_w, proj_b, codebook):
    x = ssl_content[0]
    t_len = x.shape[1]
    n_tiles = t_len // _TILE_T
    b2 = proj_b[:, None]
    return pl.pallas_call(
        _vq_block,
        grid=(n_tiles + 1,),
        in_specs=[
            pl.BlockSpec((_D, _TILE_T), lambda i: (0, jnp.minimum(i, n_tiles - 1))),
            pl.BlockSpec((_D, _D), lambda i: (0, 0)),
            pl.BlockSpec((_D, 1), lambda i: (0, 0)),
            pl.BlockSpec((_K, _D), lambda i: (0, 0)),
        ],
        out_specs=pl.BlockSpec((1, _TILE_T), lambda i: (0, jnp.maximum(i - 1, 0))),
        out_shape=jax.ShapeDtypeStruct((1, t_len), jnp.int32),
        scratch_shapes=[
            pltpu.VMEM((2, _K, _TILE_T), jnp.float32),
            pltpu.VMEM((2, 1, _TILE_T), jnp.float32),
        ],
    )(x, proj_w, b2, codebook)


# speedup vs baseline: 1.3035x; 1.3035x over previous
import jax
import jax.numpy as jnp
from jax.experimental import pallas as pl
from jax.experimental.pallas import tpu as pltpu

_D = 768
_K = 1024
_TILE_T = 512


def _vq_block(x_ref, w_ref, b_ref, cb_ref, out_ref):
    x = x_ref[...]
    w = w_ref[...]
    cb = cb_ref[...]
    y = jnp.dot(w, x, preferred_element_type=jnp.float32) + b_ref[...]
    s = jnp.dot(cb, y, preferred_element_type=jnp.float32)
    xn = jnp.sum(y * y, axis=0, keepdims=True)
    cbn = jnp.sum(cb * cb, axis=1, keepdims=True)
    dist = (xn - 2.0 * s) + cbn
    out_ref[...] = jnp.argmin(dist, axis=0)[None, :].astype(jnp.int32)


def kernel(ssl_content, proj_w, proj_b, codebook):
    x = ssl_content[0]
    t_len = x.shape[1]
    b2 = proj_b[:, None]
    return pl.pallas_call(
        _vq_block,
        grid=(t_len // _TILE_T,),
        in_specs=[
            pl.BlockSpec((_D, _TILE_T), lambda i: (0, i)),
            pl.BlockSpec((_D, _D), lambda i: (0, 0)),
            pl.BlockSpec((_D, 1), lambda i: (0, 0)),
            pl.BlockSpec((_K, _D), lambda i: (0, 0)),
        ],
        out_specs=pl.BlockSpec((1, _TILE_T), lambda i: (0, i)),
        out_shape=jax.ShapeDtypeStruct((1, t_len), jnp.int32),
        compiler_params=pltpu.CompilerParams(
            dimension_semantics=("parallel",),
        ),
    )(x, proj_w, b2, codebook)


# speedup vs baseline: 1.3845x; 1.0621x over previous
import jax
import jax.numpy as jnp
from jax.experimental import pallas as pl
from jax.experimental.pallas import tpu as pltpu

_D = 768
_K = 1024
_TILE_T = 2048


def _vq_block(x_ref, w_ref, b_ref, cb_ref, out_ref, cbn_ref):
    cb = cb_ref[...]

    @pl.when(pl.program_id(0) == 0)
    def _():
        cbn_ref[...] = jnp.sum(cb * cb, axis=1, keepdims=True)

    x = x_ref[0]
    w = w_ref[...]
    y = jnp.dot(w, x, preferred_element_type=jnp.float32) + b_ref[...]
    s = jnp.dot(cb, y, preferred_element_type=jnp.float32)
    xn = jnp.sum(y * y, axis=0, keepdims=True)
    dist = (xn - 2.0 * s) + cbn_ref[...]
    out_ref[...] = jnp.argmin(dist, axis=0)[None, :].astype(jnp.int32)


def kernel(ssl_content, proj_w, proj_b, codebook):
    t_len = ssl_content.shape[2]
    b2 = proj_b[:, None]
    return pl.pallas_call(
        _vq_block,
        grid=(t_len // _TILE_T,),
        in_specs=[
            pl.BlockSpec((1, _D, _TILE_T), lambda i: (0, 0, i)),
            pl.BlockSpec((_D, _D), lambda i: (0, 0)),
            pl.BlockSpec((_D, 1), lambda i: (0, 0)),
            pl.BlockSpec((_K, _D), lambda i: (0, 0)),
        ],
        out_specs=pl.BlockSpec((1, _TILE_T), lambda i: (0, i)),
        out_shape=jax.ShapeDtypeStruct((1, t_len), jnp.int32),
        scratch_shapes=[pltpu.VMEM((_K, 1), jnp.float32)],
    )(ssl_content, proj_w, b2, codebook)
